# baseline (speedup 1.0000x reference)
import jax
import jax.numpy as jnp
from jax.experimental import pallas as pl
from jax.experimental.pallas import tpu as pltpu

_H = 512
_W = 512
_STRIP = 8
_IPB = 8


def _lane_dilated_erosion(xs, xu):
    r = jnp.minimum(xs, xu)
    rm = jnp.concatenate([r[:, :1], r[:, :-1]], axis=1)
    rp = jnp.concatenate([r[:, 1:], r[:, _W - 2:_W - 1]], axis=1)
    return jnp.minimum(r, jnp.maximum(rm, rp))


def _dilate_sqerr(g, g_row, x):
    gd = jnp.concatenate([g[1:], g_row], axis=0)
    opened = jnp.maximum(g, gd)
    diff = x - opened
    return diff * diff


def _fold(acc, d2):
    for m in range(d2.shape[0] // 8):
        acc = acc + d2[8 * m:8 * m + 8]
    return acc


def _opening_mse_body(x_ref, out_ref):
    j = pl.program_id(0)
    n_strips = _H // _STRIP

    group = 8

    def pair_body(p, acc):
        ks = [group * p + i for i in range(group)]
        g_prev = [None] * group
        last = _STRIP - 1
        for s in range(n_strips):
            r0 = s * _STRIP
            for i, k in enumerate(ks):
                xs = x_ref[k, r0:r0 + _STRIP, :]
                if s == 0:
                    xu = jnp.concatenate([xs[0:1], xs[:-1]], axis=0)
                else:
                    xu = x_ref[k, r0 - 1:r0 + _STRIP - 1, :]
                g = _lane_dilated_erosion(xs, xu)
                if s > 0:
                    xp = x_ref[k, r0 - _STRIP:r0, :]
                    acc = _fold(acc, _dilate_sqerr(g_prev[i], g[0:1], xp))
                g_prev[i] = g
        for i, k in enumerate(ks):
            xp = x_ref[k, _H - _STRIP:_H, :]
            acc = _fold(acc, _dilate_sqerr(
                g_prev[i], g_prev[i][last:last + 1], xp))
        return acc

    acc = jax.lax.fori_loop(
        0, _IPB // group, pair_body, jnp.zeros((8, _W), jnp.float32))
    total = jnp.sum(acc).reshape(1, 1, 1)

    @pl.when(j == 0)
    def _():
        out_ref[...] = total

    @pl.when(j != 0)
    def _():
        out_ref[...] = out_ref[...] + total


def kernel(labels):
    b, c, h, w = labels.shape
    n = b * c
    x = labels.reshape(n, h, w)
    steps = n // _IPB
    partials = pl.pallas_call(
        _opening_mse_body,
        grid=(steps,),
        in_specs=[pl.BlockSpec((_IPB, h, w), lambda j: (j, 0, 0))],
        out_specs=pl.BlockSpec((1, 1, 1), lambda j: (0, 0, 0)),
        out_shape=jax.ShapeDtypeStruct((1, 1, 1), jnp.float32),
        compiler_params=pltpu.CompilerParams(
            dimension_semantics=("arbitrary",),
        ),
    )(x)
    return jnp.sum(partials) / (n * h * w)
